# baseline (speedup 1.0000x reference)
import functools

import jax
import jax.numpy as jnp
from jax import lax
from jax.experimental import pallas as pl
from jax.experimental.pallas import tpu as pltpu
from jax.experimental.pallas import tpu_sc as plsc

_B = 2
_N = 10000
_PATCH = 256
_LATENT = 256
_IDD = 64
_HID = 512
_SLOTS = 4
_M = _B * _N
_RA = 2000
_RC = 1000
_SC_N0 = 80
_SC_N1 = 80
_SC_CH = 32
_SC_NB = 10


def _bf16_bits(x):
    u = jax.lax.bitcast_convert_type(x, jnp.uint32)
    return (u + jnp.uint32(0x7FFF) + ((u >> 16) & jnp.uint32(1))) >> 16


def _sender_body(z_ref, id_ref, w1z_ref, w1sid_ref, s_ref):
    f32 = jnp.float32
    s = (jnp.dot(z_ref[...], w1z_ref[...], preferred_element_type=f32)
         + jnp.dot(id_ref[...], w1sid_ref[...], preferred_element_type=f32))
    hw = _HID // 2
    lo = _bf16_bits(s[:, :hw])
    hi = _bf16_bits(s[:, hw:])
    s_ref[...] = jax.lax.bitcast_convert_type(lo | (hi << 16), f32)


def _obs_body(obs_ref, omask_ref,
              wo0_ref, bo0_ref, wo1_ref, bo1_ref, wo2_ref, bo2_ref, oe_ref):
    f32 = jnp.float32
    h = jnp.maximum(jnp.dot(obs_ref[...], wo0_ref[...], preferred_element_type=f32)
                    + bo0_ref[...], 0.0)
    h = jnp.maximum(jnp.dot(h, wo1_ref[...], preferred_element_type=f32)
                    + bo1_ref[...], 0.0)
    oe = jnp.dot(h, wo2_ref[...], preferred_element_type=f32) + bo2_ref[...]
    oe_ref[...] = oe * omask_ref[...]


def _phase_c_body(g_ref, oe_ref, z_ref, id_ref, ed_ref, nm_ref,
                  omask_ref, w1id_ref, b1_ref, wrel_ref, w2_ref, b2_ref,
                  wiho_ref, wihm_ref, wihi_ref, wihmk_ref, bih_ref,
                  whh_ref, bhh_ref,
                  ws0_ref, bs0_ref, ws1_ref, bs1_ref, ws2_ref, bs2_ref,
                  wn0_ref, bn0_ref, wn1_ref, bn1_ref, wn2_ref, bn2_ref,
                  zn_ref, sp_ref, npred_ref):
    f32 = jnp.float32
    L = _LATENT
    rv = (jnp.dot(id_ref[...].astype(jnp.bfloat16), w1id_ref[...],
                  preferred_element_type=f32) + b1_ref[...])
    ed = ed_ref[...]
    nm = nm_ref[...]
    hw = _HID // 2
    rows = g_ref.shape[1]
    bf = jnp.bfloat16
    hlo = jnp.zeros((rows, hw), f32)
    hhi = jnp.zeros((rows, hw), f32)
    for s in range(_SLOTS):
        elo = (ed[:, 2 * s:2 * s + 1] * wrel_ref[0:1, :hw]
               + ed[:, 2 * s + 1:2 * s + 2] * wrel_ref[1:2, :hw])
        ehi = (ed[:, 2 * s:2 * s + 1] * wrel_ref[0:1, hw:]
               + ed[:, 2 * s + 1:2 * s + 2] * wrel_ref[1:2, hw:])
        w = jax.lax.bitcast_convert_type(g_ref[s], jnp.uint32)
        glo = jax.lax.bitcast_convert_type(w << 16, f32)
        ghi = jax.lax.bitcast_convert_type(w & jnp.uint32(0xFFFF0000), f32)
        nms = nm[:, s:s + 1]
        hlo = hlo + nms * jnp.maximum(glo + rv[:, :hw] + elo, 0.0)
        hhi = hhi + nms * jnp.maximum(ghi + rv[:, hw:] + ehi, 0.0)
    msum = jnp.sum(nm, axis=1, keepdims=True)
    agg = (jnp.dot(hlo.astype(bf), w2_ref[:hw], preferred_element_type=f32)
           + jnp.dot(hhi.astype(bf), w2_ref[hw:], preferred_element_type=f32)
           + msum * b2_ref[...])
    msg = agg / jnp.maximum(msum, 1.0)
    z_prev = z_ref[...]
    gi = (jnp.dot(oe_ref[...].astype(bf), wiho_ref[...], preferred_element_type=f32)
          + jnp.dot(msg.astype(bf), wihm_ref[...], preferred_element_type=f32)
          + jnp.dot(id_ref[...].astype(bf), wihi_ref[...], preferred_element_type=f32)
          + omask_ref[...] * wihmk_ref[...] + bih_ref[...])
    gh = (jnp.dot(z_prev.astype(bf), whh_ref[...], preferred_element_type=f32)
          + bhh_ref[...])
    r = jax.nn.sigmoid(gi[:, :L] + gh[:, :L])
    zg = jax.nn.sigmoid(gi[:, L:2 * L] + gh[:, L:2 * L])
    n = jnp.tanh(gi[:, 2 * L:] + r * gh[:, 2 * L:])
    zn = (1.0 - zg) * n + zg * z_prev
    zn_ref[...] = zn
    znb = zn.astype(bf)
    t = jnp.maximum(jnp.dot(znb, ws0_ref[...], preferred_element_type=f32) + bs0_ref[...], 0.0)
    t = jnp.maximum(jnp.dot(t.astype(bf), ws1_ref[...], preferred_element_type=f32) + bs1_ref[...], 0.0)
    sp_ref[...] = jnp.dot(t.astype(bf), ws2_ref[...], preferred_element_type=f32) + bs2_ref[...]
    t = jnp.maximum(jnp.dot(znb, wn0_ref[...], preferred_element_type=f32) + bn0_ref[...], 0.0)
    t = jnp.maximum(jnp.dot(t.astype(bf), wn1_ref[...], preferred_element_type=f32) + bn1_ref[...], 0.0)
    tb = t.astype(bf)
    for s in range(_SLOTS):
        npred_ref[:, s, :] = (
            jnp.dot(tb, wn2_ref[:, s * L:(s + 1) * L], preferred_element_type=f32)
            + bn2_ref[:, s * L:(s + 1) * L])


def _gather_sc(table, idx):
    info = plsc.get_sparse_core_info()
    ch = idx.shape[1]
    n0, n1 = _SC_N0, _SC_N1
    t_rows = info.num_subcores * (n0 + n1) * ch
    nb = _SC_NB
    mesh = plsc.VectorSubcoreMesh(core_axis_name="c", subcore_axis_name="s")

    @functools.partial(
        pl.kernel, mesh=mesh,
        out_type=jax.ShapeDtypeStruct((t_rows, _HID // 2), jnp.float32),
        scratch_types=[pltpu.VMEM((n0, ch), jnp.int32),
                       pltpu.VMEM((nb, ch, _HID // 2), jnp.float32)]
                      + [pltpu.SemaphoreType.DMA] * (2 * nb))
    def k(table_hbm, idx_hbm, out_hbm, idx_v, buf_v, *sems):
        gsems, wsems = sems[:nb], sems[nb:]
        s_ax = lax.axis_index("s")
        c_ax = lax.axis_index("c")
        nchunk = n0 if n0 == n1 else jnp.where(c_ax == 0, n0, n1)
        base_row = s_ax * (n0 + n1) + c_ax * n0
        base = base_row * ch
        pltpu.sync_copy(idx_hbm.at[pl.ds(base_row, n0)], idx_v)

        def start_g(c, b):
            pltpu.async_copy(table_hbm.at[idx_v.at[c]], buf_v.at[b], gsems[b])

        def wait_g(b):
            pltpu.make_async_copy(table_hbm.at[pl.ds(0, ch)],
                                  buf_v.at[b], gsems[b]).wait()

        def start_w(c, b):
            pltpu.async_copy(buf_v.at[b],
                             out_hbm.at[pl.ds(base + c * ch, ch)], wsems[b])

        def wait_w(b):
            pltpu.make_async_copy(buf_v.at[b],
                                  out_hbm.at[pl.ds(0, ch)], wsems[b]).wait()

        for b in range(nb):
            start_g(b, b)

        def group(gidx, carry):
            for b in range(nb):
                c = gidx * nb + b
                wait_g(b)
                start_w(c, b)
            for b in range(nb):
                c = gidx * nb + b

                @pl.when(c + nb < nchunk)
                def _():
                    wait_w(b)
                    start_g(c + nb, b)
            return carry

        lax.fori_loop(0, nchunk // nb, group, 0)
        for b in range(nb):
            wait_w(b)

    return k(table, idx)


def kernel(z_prev, obs_patches, obs_mask, id_features, neighbor_idx,
           neighbor_mask, edge_delta, disable_messages, params):
    f32 = jnp.float32
    (wo0, bo0), (wo1, bo1), (wo2, bo2) = params["obs"]
    (wm1, bm1), (wm2, bm2) = params["msg"]
    wih, bih, whh, bhh = params["gru"]
    (ws0, bs0), (ws1, bs1), (ws2, bs2) = params["self"]
    (wn0, bn0), (wn1, bn1), (wn2, bn2) = params["nb"]

    w1z = wm1[:, :_LATENT].T
    w1sid = wm1[:, _LATENT:_LATENT + _IDD].T
    w1id = wm1[:, _LATENT + _IDD:_LATENT + 2 * _IDD].T
    wrel = wm1[:, _LATENT + 2 * _IDD:]
    wrel = wrel.T
    wiho = wih[:, :_LATENT].T
    wihm = wih[:, _LATENT:2 * _LATENT].T
    wihi = wih[:, 2 * _LATENT:2 * _LATENT + _IDD].T
    wihmk = wih[:, 2 * _LATENT + _IDD].reshape(1, 3 * _LATENT)

    zf = z_prev.reshape(_M, _LATENT)
    obsf = obs_patches.reshape(_M, _PATCH)
    idf = id_features.reshape(_M, _IDD)
    omaskf = obs_mask.reshape(_M, 1)

    row2 = lambda v: v.reshape(1, -1)
    grid_a = _M // _RA
    full = lambda shp: pl.BlockSpec(shp, lambda i: (0, 0))
    rowblk = lambda d, r: pl.BlockSpec((r, d), lambda i: (i, 0))
    s_out = pl.pallas_call(
        _sender_body,
        grid=(grid_a,),
        in_specs=[rowblk(_LATENT, _RA), rowblk(_IDD, _RA),
                  full((_LATENT, _HID)), full((_IDD, _HID))],
        out_specs=rowblk(_HID // 2, _RA),
        out_shape=jax.ShapeDtypeStruct((_M, _HID // 2), f32),
    )(zf, idf, w1z, w1sid)

    oe_out = pl.pallas_call(
        _obs_body,
        grid=(grid_a,),
        in_specs=[rowblk(_PATCH, _RA), rowblk(1, _RA),
                  full((_PATCH, _HID)), full((1, _HID)),
                  full((_HID, _HID)), full((1, _HID)),
                  full((_HID, _LATENT)), full((1, _LATENT))],
        out_specs=rowblk(_LATENT, _RA),
        out_shape=jax.ShapeDtypeStruct((_M, _LATENT), f32),
    )(obsf, omaskf, wo0.T, row2(bo0), wo1.T, row2(bo1), wo2.T, row2(bo2))

    mp = 20480
    idx_c = jnp.maximum(neighbor_idx, 0)
    boff = (jnp.arange(_B, dtype=jnp.int32) * _N)[:, None]
    cols = [jnp.pad((boff + idx_c[:, s][None, :]).reshape(_M), (0, mp - _M))
            for s in range(_SLOTS)]
    flat_idx = jnp.concatenate(cols, axis=0).astype(jnp.int32)
    nrow = _SLOTS * mp // _SC_CH
    flat_idx = jnp.pad(flat_idx, (0, (_SC_N0 - _SC_N1) * _SC_CH)).reshape(
        nrow + _SC_N0 - _SC_N1, _SC_CH)

    g = _gather_sc(s_out, flat_idx).reshape(_SLOTS, mp, _HID // 2)

    bfc = lambda w: w.astype(jnp.bfloat16)
    scale = (jnp.asarray(disable_messages) == 0).astype(f32)
    nmf = neighbor_mask * scale
    edf = edge_delta.reshape(_N, 2 * _SLOTS)

    grid_c = _M // _RC
    nblk = _N // _RC
    gblk = pl.BlockSpec((_SLOTS, _RC, _HID // 2), lambda i: (0, i, 0))
    rowblk_c = lambda d: pl.BlockSpec((_RC, d), lambda i: (i, 0))
    nrowblk = lambda d: pl.BlockSpec((_RC, d), lambda i: (i % nblk, 0))
    zn, sp, npred = pl.pallas_call(
        _phase_c_body,
        grid=(grid_c,),
        in_specs=[gblk, rowblk_c(_LATENT), rowblk_c(_LATENT),
                  rowblk_c(_IDD), nrowblk(2 * _SLOTS), nrowblk(_SLOTS),
                  rowblk_c(1),
                  full((_IDD, _HID)), full((1, _HID)),
                  full((2, _HID)), full((_HID, _LATENT)), full((1, _LATENT)),
                  full((_LATENT, 3 * _LATENT)), full((_LATENT, 3 * _LATENT)),
                  full((_IDD, 3 * _LATENT)), full((1, 3 * _LATENT)),
                  full((1, 3 * _LATENT)),
                  full((_LATENT, 3 * _LATENT)), full((1, 3 * _LATENT)),
                  full((_LATENT, _HID)), full((1, _HID)),
                  full((_HID, _HID)), full((1, _HID)),
                  full((_HID, _PATCH)), full((1, _PATCH)),
                  full((_LATENT, _HID)), full((1, _HID)),
                  full((_HID, _HID)), full((1, _HID)),
                  full((_HID, _SLOTS * _LATENT)), full((1, _SLOTS * _LATENT))],
        out_specs=[rowblk_c(_LATENT), rowblk_c(_PATCH),
                   pl.BlockSpec((_RC, _SLOTS, _LATENT), lambda i: (i, 0, 0))],
        out_shape=[jax.ShapeDtypeStruct((_M, _LATENT), f32),
                   jax.ShapeDtypeStruct((_M, _PATCH), f32),
                   jax.ShapeDtypeStruct((_M, _SLOTS, _LATENT), f32)],
    )(g, oe_out, zf, idf, edf, nmf, omaskf,
      bfc(w1id), row2(bm1), wrel, bfc(wm2.T), row2(bm2),
      bfc(wiho), bfc(wihm), bfc(wihi), wihmk, row2(bih),
      bfc(whh.T), row2(bhh),
      bfc(ws0.T), row2(bs0), bfc(ws1.T), row2(bs1), bfc(ws2.T), row2(bs2),
      bfc(wn0.T), row2(bn0), bfc(wn1.T), row2(bn1), bfc(wn2.T), row2(bn2))

    z_next = zn.reshape(_B, _N, _LATENT)
    self_pred = sp.reshape(_B, _N, _PATCH)
    neighbor_pred = npred.reshape(_B, _N, _SLOTS, _LATENT)
    return (z_next, self_pred, neighbor_pred)


# speedup vs baseline: 1.0042x; 1.0042x over previous
import functools

import jax
import jax.numpy as jnp
from jax import lax
from jax.experimental import pallas as pl
from jax.experimental.pallas import tpu as pltpu
from jax.experimental.pallas import tpu_sc as plsc

_B = 2
_N = 10000
_PATCH = 256
_LATENT = 256
_IDD = 64
_HID = 512
_SLOTS = 4
_M = _B * _N
_RA = 2000
_RC = 1000
_SC_N0 = 80
_SC_N1 = 80
_SC_CH = 32
_SC_NB = 8


def _bf16_bits(x):
    u = jax.lax.bitcast_convert_type(x, jnp.uint32)
    return (u + jnp.uint32(0x7FFF) + ((u >> 16) & jnp.uint32(1))) >> 16


def _sender_body(z_ref, id_ref, w1z_ref, w1sid_ref, s_ref):
    f32 = jnp.float32
    s = (jnp.dot(z_ref[...], w1z_ref[...], preferred_element_type=f32)
         + jnp.dot(id_ref[...], w1sid_ref[...], preferred_element_type=f32))
    hw = _HID // 2
    lo = _bf16_bits(s[:, :hw])
    hi = _bf16_bits(s[:, hw:])
    s_ref[...] = jax.lax.bitcast_convert_type(lo | (hi << 16), f32)


def _obs_body(obs_ref, omask_ref,
              wo0_ref, bo0_ref, wo1_ref, bo1_ref, wo2_ref, bo2_ref, oe_ref):
    f32 = jnp.float32
    h = jnp.maximum(jnp.dot(obs_ref[...], wo0_ref[...], preferred_element_type=f32)
                    + bo0_ref[...], 0.0)
    h = jnp.maximum(jnp.dot(h, wo1_ref[...], preferred_element_type=f32)
                    + bo1_ref[...], 0.0)
    oe = jnp.dot(h, wo2_ref[...], preferred_element_type=f32) + bo2_ref[...]
    oe_ref[...] = oe * omask_ref[...]


def _phase_c_body(g_ref, oe_ref, z_ref, id_ref, ed_ref, nm_ref,
                  omask_ref, w1id_ref, b1_ref, wrel_ref, w2_ref, b2_ref,
                  wiho_ref, wihm_ref, wihi_ref, wihmk_ref, bih_ref,
                  whh_ref, bhh_ref,
                  ws0_ref, bs0_ref, ws1_ref, bs1_ref, ws2_ref, bs2_ref,
                  wn0_ref, bn0_ref, wn1_ref, bn1_ref, wn2_ref, bn2_ref,
                  zn_ref, sp_ref, npred_ref):
    f32 = jnp.float32
    L = _LATENT
    rv = (jnp.dot(id_ref[...].astype(jnp.bfloat16), w1id_ref[...],
                  preferred_element_type=f32) + b1_ref[...])
    ed = ed_ref[...]
    nm = nm_ref[...]
    hw = _HID // 2
    rows = g_ref.shape[1]
    bf = jnp.bfloat16
    hlo = jnp.zeros((rows, hw), f32)
    hhi = jnp.zeros((rows, hw), f32)
    for s in range(_SLOTS):
        elo = (ed[:, 2 * s:2 * s + 1] * wrel_ref[0:1, :hw]
               + ed[:, 2 * s + 1:2 * s + 2] * wrel_ref[1:2, :hw])
        ehi = (ed[:, 2 * s:2 * s + 1] * wrel_ref[0:1, hw:]
               + ed[:, 2 * s + 1:2 * s + 2] * wrel_ref[1:2, hw:])
        w = jax.lax.bitcast_convert_type(g_ref[s], jnp.uint32)
        glo = jax.lax.bitcast_convert_type(w << 16, f32)
        ghi = jax.lax.bitcast_convert_type(w & jnp.uint32(0xFFFF0000), f32)
        nms = nm[:, s:s + 1]
        hlo = hlo + nms * jnp.maximum(glo + rv[:, :hw] + elo, 0.0)
        hhi = hhi + nms * jnp.maximum(ghi + rv[:, hw:] + ehi, 0.0)
    msum = jnp.sum(nm, axis=1, keepdims=True)
    agg = (jnp.dot(hlo.astype(bf), w2_ref[:hw], preferred_element_type=f32)
           + jnp.dot(hhi.astype(bf), w2_ref[hw:], preferred_element_type=f32)
           + msum * b2_ref[...])
    msg = agg / jnp.maximum(msum, 1.0)
    z_prev = z_ref[...]
    gi = (jnp.dot(oe_ref[...].astype(bf), wiho_ref[...], preferred_element_type=f32)
          + jnp.dot(msg.astype(bf), wihm_ref[...], preferred_element_type=f32)
          + jnp.dot(id_ref[...].astype(bf), wihi_ref[...], preferred_element_type=f32)
          + omask_ref[...] * wihmk_ref[...] + bih_ref[...])
    gh = (jnp.dot(z_prev.astype(bf), whh_ref[...], preferred_element_type=f32)
          + bhh_ref[...])
    r = jax.nn.sigmoid(gi[:, :L] + gh[:, :L])
    zg = jax.nn.sigmoid(gi[:, L:2 * L] + gh[:, L:2 * L])
    n = jnp.tanh(gi[:, 2 * L:] + r * gh[:, 2 * L:])
    zn = (1.0 - zg) * n + zg * z_prev
    zn_ref[...] = zn
    znb = zn.astype(bf)
    t = jnp.maximum(jnp.dot(znb, ws0_ref[...], preferred_element_type=f32) + bs0_ref[...], 0.0)
    t = jnp.maximum(jnp.dot(t.astype(bf), ws1_ref[...], preferred_element_type=f32) + bs1_ref[...], 0.0)
    sp_ref[...] = jnp.dot(t.astype(bf), ws2_ref[...], preferred_element_type=f32) + bs2_ref[...]
    t = jnp.maximum(jnp.dot(znb, wn0_ref[...], preferred_element_type=f32) + bn0_ref[...], 0.0)
    t = jnp.maximum(jnp.dot(t.astype(bf), wn1_ref[...], preferred_element_type=f32) + bn1_ref[...], 0.0)
    tb = t.astype(bf)
    for s in range(_SLOTS):
        npred_ref[:, s, :] = (
            jnp.dot(tb, wn2_ref[:, s * L:(s + 1) * L], preferred_element_type=f32)
            + bn2_ref[:, s * L:(s + 1) * L])


def _gather_sc(table, idx):
    info = plsc.get_sparse_core_info()
    ch = idx.shape[1]
    n0, n1 = _SC_N0, _SC_N1
    t_rows = info.num_subcores * (n0 + n1) * ch
    nb = _SC_NB
    mesh = plsc.VectorSubcoreMesh(core_axis_name="c", subcore_axis_name="s")

    @functools.partial(
        pl.kernel, mesh=mesh,
        out_type=jax.ShapeDtypeStruct((t_rows, _HID // 2), jnp.float32),
        scratch_types=[pltpu.VMEM((n0, ch), jnp.int32),
                       pltpu.VMEM((nb, ch, _HID // 2), jnp.float32)]
                      + [pltpu.SemaphoreType.DMA] * (2 * nb))
    def k(table_hbm, idx_hbm, out_hbm, idx_v, buf_v, *sems):
        gsems, wsems = sems[:nb], sems[nb:]
        s_ax = lax.axis_index("s")
        c_ax = lax.axis_index("c")
        nchunk = n0 if n0 == n1 else jnp.where(c_ax == 0, n0, n1)
        base_row = s_ax * (n0 + n1) + c_ax * n0
        base = base_row * ch
        pltpu.sync_copy(idx_hbm.at[pl.ds(base_row, n0)], idx_v)

        def start_g(c, b):
            pltpu.async_copy(table_hbm.at[idx_v.at[c]], buf_v.at[b], gsems[b])

        def wait_g(b):
            pltpu.make_async_copy(table_hbm.at[pl.ds(0, ch)],
                                  buf_v.at[b], gsems[b]).wait()

        def start_w(c, b):
            pltpu.async_copy(buf_v.at[b],
                             out_hbm.at[pl.ds(base + c * ch, ch)], wsems[b])

        def wait_w(b):
            pltpu.make_async_copy(buf_v.at[b],
                                  out_hbm.at[pl.ds(0, ch)], wsems[b]).wait()

        for b in range(nb):
            start_g(b, b)

        def group(gidx, carry):
            for b in range(nb):
                c = gidx * nb + b
                wait_g(b)
                start_w(c, b)
            for b in range(nb):
                c = gidx * nb + b

                @pl.when(c + nb < nchunk)
                def _():
                    wait_w(b)
                    start_g(c + nb, b)
            return carry

        lax.fori_loop(0, nchunk // nb, group, 0)
        for b in range(nb):
            wait_w(b)

    return k(table, idx)


def kernel(z_prev, obs_patches, obs_mask, id_features, neighbor_idx,
           neighbor_mask, edge_delta, disable_messages, params):
    f32 = jnp.float32
    (wo0, bo0), (wo1, bo1), (wo2, bo2) = params["obs"]
    (wm1, bm1), (wm2, bm2) = params["msg"]
    wih, bih, whh, bhh = params["gru"]
    (ws0, bs0), (ws1, bs1), (ws2, bs2) = params["self"]
    (wn0, bn0), (wn1, bn1), (wn2, bn2) = params["nb"]

    w1z = wm1[:, :_LATENT].T
    w1sid = wm1[:, _LATENT:_LATENT + _IDD].T
    w1id = wm1[:, _LATENT + _IDD:_LATENT + 2 * _IDD].T
    wrel = wm1[:, _LATENT + 2 * _IDD:]
    wrel = wrel.T
    wiho = wih[:, :_LATENT].T
    wihm = wih[:, _LATENT:2 * _LATENT].T
    wihi = wih[:, 2 * _LATENT:2 * _LATENT + _IDD].T
    wihmk = wih[:, 2 * _LATENT + _IDD].reshape(1, 3 * _LATENT)

    zf = z_prev.reshape(_M, _LATENT)
    obsf = obs_patches.reshape(_M, _PATCH)
    idf = id_features.reshape(_M, _IDD)
    omaskf = obs_mask.reshape(_M, 1)

    row2 = lambda v: v.reshape(1, -1)
    grid_a = _M // _RA
    full = lambda shp: pl.BlockSpec(shp, lambda i: (0, 0))
    rowblk = lambda d, r: pl.BlockSpec((r, d), lambda i: (i, 0))
    s_out = pl.pallas_call(
        _sender_body,
        grid=(grid_a,),
        in_specs=[rowblk(_LATENT, _RA), rowblk(_IDD, _RA),
                  full((_LATENT, _HID)), full((_IDD, _HID))],
        out_specs=rowblk(_HID // 2, _RA),
        out_shape=jax.ShapeDtypeStruct((_M, _HID // 2), f32),
    )(zf, idf, w1z, w1sid)

    oe_out = pl.pallas_call(
        _obs_body,
        grid=(grid_a,),
        in_specs=[rowblk(_PATCH, _RA), rowblk(1, _RA),
                  full((_PATCH, _HID)), full((1, _HID)),
                  full((_HID, _HID)), full((1, _HID)),
                  full((_HID, _LATENT)), full((1, _LATENT))],
        out_specs=rowblk(_LATENT, _RA),
        out_shape=jax.ShapeDtypeStruct((_M, _LATENT), f32),
    )(obsf, omaskf, wo0.T, row2(bo0), wo1.T, row2(bo1), wo2.T, row2(bo2))

    mp = 20480
    idx_c = jnp.maximum(neighbor_idx, 0)
    boff = (jnp.arange(_B, dtype=jnp.int32) * _N)[:, None]
    cols = [jnp.pad((boff + idx_c[:, s][None, :]).reshape(_M), (0, mp - _M))
            for s in range(_SLOTS)]
    flat_idx = jnp.concatenate(cols, axis=0).astype(jnp.int32)
    nrow = _SLOTS * mp // _SC_CH
    flat_idx = jnp.pad(flat_idx, (0, (_SC_N0 - _SC_N1) * _SC_CH)).reshape(
        nrow + _SC_N0 - _SC_N1, _SC_CH)

    g = _gather_sc(s_out, flat_idx).reshape(_SLOTS, mp, _HID // 2)

    bfc = lambda w: w.astype(jnp.bfloat16)
    scale = (jnp.asarray(disable_messages) == 0).astype(f32)
    nmf = neighbor_mask * scale
    edf = edge_delta.reshape(_N, 2 * _SLOTS)

    grid_c = _M // _RC
    nblk = _N // _RC
    gblk = pl.BlockSpec((_SLOTS, _RC, _HID // 2), lambda i: (0, i, 0))
    rowblk_c = lambda d: pl.BlockSpec((_RC, d), lambda i: (i, 0))
    nrowblk = lambda d: pl.BlockSpec((_RC, d), lambda i: (i % nblk, 0))
    zn, sp, npred = pl.pallas_call(
        _phase_c_body,
        grid=(grid_c,),
        in_specs=[gblk, rowblk_c(_LATENT), rowblk_c(_LATENT),
                  rowblk_c(_IDD), nrowblk(2 * _SLOTS), nrowblk(_SLOTS),
                  rowblk_c(1),
                  full((_IDD, _HID)), full((1, _HID)),
                  full((2, _HID)), full((_HID, _LATENT)), full((1, _LATENT)),
                  full((_LATENT, 3 * _LATENT)), full((_LATENT, 3 * _LATENT)),
                  full((_IDD, 3 * _LATENT)), full((1, 3 * _LATENT)),
                  full((1, 3 * _LATENT)),
                  full((_LATENT, 3 * _LATENT)), full((1, 3 * _LATENT)),
                  full((_LATENT, _HID)), full((1, _HID)),
                  full((_HID, _HID)), full((1, _HID)),
                  full((_HID, _PATCH)), full((1, _PATCH)),
                  full((_LATENT, _HID)), full((1, _HID)),
                  full((_HID, _HID)), full((1, _HID)),
                  full((_HID, _SLOTS * _LATENT)), full((1, _SLOTS * _LATENT))],
        out_specs=[rowblk_c(_LATENT), rowblk_c(_PATCH),
                   pl.BlockSpec((_RC, _SLOTS, _LATENT), lambda i: (i, 0, 0))],
        out_shape=[jax.ShapeDtypeStruct((_M, _LATENT), f32),
                   jax.ShapeDtypeStruct((_M, _PATCH), f32),
                   jax.ShapeDtypeStruct((_M, _SLOTS, _LATENT), f32)],
    )(g, oe_out, zf, idf, edf, nmf, omaskf,
      bfc(w1id), row2(bm1), wrel, bfc(wm2.T), row2(bm2),
      bfc(wiho), bfc(wihm), bfc(wihi), wihmk, row2(bih),
      bfc(whh.T), row2(bhh),
      bfc(ws0.T), row2(bs0), bfc(ws1.T), row2(bs1), bfc(ws2.T), row2(bs2),
      bfc(wn0.T), row2(bn0), bfc(wn1.T), row2(bn1), bfc(wn2.T), row2(bn2))

    z_next = zn.reshape(_B, _N, _LATENT)
    self_pred = sp.reshape(_B, _N, _PATCH)
    neighbor_pred = npred.reshape(_B, _N, _SLOTS, _LATENT)
    return (z_next, self_pred, neighbor_pred)
